# baseline (speedup 1.0000x reference)
import functools

import jax
import jax.numpy as jnp
from jax import lax
from jax.experimental import pallas as pl
from jax.experimental.pallas import tpu as pltpu
from jax.experimental.pallas import tpu_sc as plsc


def _sc_row_gather(table, idx, out_rows):
    info = plsc.get_sparse_core_info()
    nw = info.num_cores * info.num_subcores
    d = table.shape[1]
    b_per_w = out_rows // nw
    mesh = plsc.VectorSubcoreMesh(core_axis_name="c", subcore_axis_name="s")

    @functools.partial(
        pl.kernel,
        mesh=mesh,
        out_type=jax.ShapeDtypeStruct((out_rows, d), table.dtype),
        scratch_types=[
            pltpu.VMEM((b_per_w,), jnp.int32),
            pltpu.VMEM((b_per_w, d), table.dtype),
            pltpu.SemaphoreType.DMA,
        ],
    )
    def k(table_hbm, idx_hbm, out_hbm, idx_v, rows_v, sem):
        wid = lax.axis_index("s") * info.num_cores + lax.axis_index("c")
        base = wid * b_per_w
        pltpu.sync_copy(idx_hbm.at[pl.ds(base, b_per_w)], idx_v)
        pltpu.async_copy(table_hbm.at[idx_v], rows_v, sem).wait()
        pltpu.sync_copy(rows_v, out_hbm.at[pl.ds(base, b_per_w)])

    return k(table, idx)


def _grouped_mm_body(num_experts, tile_m, out_size, offs_ref, x_ref, w_ref, out_ref):
    t = pl.program_id(0)
    r0 = t * tile_m
    for e in range(num_experts):
        seg_lo = offs_ref[e]
        seg_hi = offs_ref[e + 1]

        @pl.when((seg_lo < r0 + tile_m) & (seg_hi > r0))
        def _():
            sub = lax.dot_general(
                x_ref[...],
                w_ref[e],
                (((1,), (1,)), ((), ())),
                preferred_element_type=jnp.float32,
            )
            rows = r0 + lax.broadcasted_iota(jnp.int32, (tile_m, out_size), 0)
            m = (rows >= seg_lo) & (rows < seg_hi)
            out_ref[...] = jnp.where(m, sub, out_ref[...])


def kernel(batch, classes, W):
    b, in_size = batch.shape
    e, out_size, _ = W.shape
    clz = classes.astype(jnp.int32)

    onehot = (clz[:, None] == jnp.arange(e, dtype=jnp.int32)[None, :]).astype(
        jnp.int32
    )
    cum = jnp.cumsum(onehot, axis=0)
    counts = cum[-1]
    offs = jnp.concatenate(
        [jnp.zeros((1,), jnp.int32), jnp.cumsum(counts, dtype=jnp.int32)]
    )
    rank = cum - onehot
    pos = jnp.sum(
        onehot * (offs[:e][None, :] + rank), axis=1, dtype=jnp.int32
    )
    sort_idx = jnp.zeros((b,), jnp.int32).at[pos].set(
        jnp.arange(b, dtype=jnp.int32)
    )

    sorted_x = _sc_row_gather(batch, sort_idx, b)

    tile_m = 256
    body = functools.partial(_grouped_mm_body, e, tile_m, out_size)
    sorted_out = pl.pallas_call(
        body,
        grid=(b // tile_m,),
        in_specs=[
            pl.BlockSpec(memory_space=pltpu.SMEM),
            pl.BlockSpec((tile_m, in_size), lambda t: (t, 0)),
            pl.BlockSpec((e, out_size, in_size), lambda t: (0, 0, 0)),
        ],
        out_specs=pl.BlockSpec((tile_m, out_size), lambda t: (t, 0)),
        out_shape=jax.ShapeDtypeStruct((b, out_size), jnp.float32),
    )(offs, sorted_x, W)

    return _sc_row_gather(sorted_out, pos, b)


# speedup vs baseline: 1.1342x; 1.1342x over previous
import functools

import jax
import jax.numpy as jnp
from jax import lax
from jax.experimental import pallas as pl
from jax.experimental.pallas import tpu as pltpu
from jax.experimental.pallas import tpu_sc as plsc


def _sc_row_scatter(rows, pos, n_rows):
    info = plsc.get_sparse_core_info()
    nw = info.num_cores * info.num_subcores
    d = rows.shape[1]
    b_per_w = n_rows // nw
    mesh = plsc.VectorSubcoreMesh(core_axis_name="c", subcore_axis_name="s")

    @functools.partial(
        pl.kernel,
        mesh=mesh,
        out_type=jax.ShapeDtypeStruct((n_rows, d), rows.dtype),
        scratch_types=[
            pltpu.VMEM((b_per_w,), jnp.int32),
            pltpu.VMEM((b_per_w, d), rows.dtype),
            pltpu.SemaphoreType.DMA,
        ],
    )
    def k(rows_hbm, pos_hbm, out_hbm, idx_v, rows_v, sem):
        wid = lax.axis_index("s") * info.num_cores + lax.axis_index("c")
        base = wid * b_per_w
        pltpu.sync_copy(pos_hbm.at[pl.ds(base, b_per_w)], idx_v)
        pltpu.sync_copy(rows_hbm.at[pl.ds(base, b_per_w)], rows_v)
        pltpu.async_copy(rows_v, out_hbm.at[idx_v], sem).wait()

    return k(rows, pos)


def _fused_body(
    n_tiles, tile_m, num_experts, out_size, b,
    offs_ref, x_ref, w_ref, pos_ref, out_ref, smm_ref, acc_ref,
):
    t = pl.program_id(0)

    @pl.when(t < n_tiles)
    def _mm():
        r0 = t * tile_m
        for e in range(num_experts):
            seg_lo = offs_ref[e]
            seg_hi = offs_ref[e + 1]

            @pl.when((seg_lo < r0 + tile_m) & (seg_hi > r0))
            def _():
                sub = lax.dot_general(
                    x_ref[...],
                    w_ref[e],
                    (((1,), (1,)), ((), ())),
                    preferred_element_type=jnp.float32,
                )
                rows = r0 + lax.broadcasted_iota(jnp.int32, (tile_m, out_size), 0)
                m = (rows >= seg_lo) & (rows < seg_hi)
                acc_ref[...] = jnp.where(m, sub, acc_ref[...])

        smm_ref[pl.ds(r0, tile_m), :] = acc_ref[...].astype(jnp.bfloat16)

    @pl.when(t >= n_tiles)
    def _unpermute():
        prow = pos_ref[0]
        eye = (
            lax.broadcasted_iota(jnp.int32, (tile_m, tile_m), 0)
            == lax.broadcasted_iota(jnp.int32, (tile_m, tile_m), 1)
        ).astype(jnp.float32)
        pcol = lax.dot_general(
            eye, prow, (((1,), (1,)), ((), ())),
            preferred_element_type=jnp.float32,
        )
        slots = lax.broadcasted_iota(jnp.int32, (tile_m, b), 1).astype(jnp.float32)
        onehot = (slots == pcol).astype(jnp.bfloat16)
        out_ref[...] = lax.dot_general(
            onehot, smm_ref[...], (((1,), (0,)), ((), ())),
            preferred_element_type=jnp.float32,
        )


def kernel(batch, classes, W):
    b, in_size = batch.shape
    e, out_size, _ = W.shape
    clz = classes.astype(jnp.int32)

    onehot = (clz[:, None] == jnp.arange(e, dtype=jnp.int32)[None, :]).astype(
        jnp.int32
    )
    cum = jnp.cumsum(onehot, axis=0)
    counts = cum[-1]
    offs = jnp.concatenate(
        [jnp.zeros((1,), jnp.int32), jnp.cumsum(counts, dtype=jnp.int32)]
    )
    rank = cum - onehot
    pos = jnp.sum(
        onehot * (offs[:e][None, :] + rank), axis=1, dtype=jnp.int32
    )

    sorted_x = _sc_row_scatter(batch, pos, b)

    tile_m = 256
    n_tiles = b // tile_m
    pos_f = pos.astype(jnp.float32).reshape(n_tiles, 1, tile_m)
    body = functools.partial(_fused_body, n_tiles, tile_m, e, out_size, b)
    return pl.pallas_call(
        body,
        grid=(2 * n_tiles,),
        in_specs=[
            pl.BlockSpec(memory_space=pltpu.SMEM),
            pl.BlockSpec(
                (tile_m, in_size), lambda t: (jnp.minimum(t, n_tiles - 1), 0)
            ),
            pl.BlockSpec((e, out_size, in_size), lambda t: (0, 0, 0)),
            pl.BlockSpec(
                (1, 1, tile_m), lambda t: (jnp.maximum(t - n_tiles, 0), 0, 0)
            ),
        ],
        out_specs=pl.BlockSpec(
            (tile_m, out_size), lambda t: (jnp.maximum(t - n_tiles, 0), 0)
        ),
        out_shape=jax.ShapeDtypeStruct((b, out_size), jnp.float32),
        scratch_shapes=[
            pltpu.VMEM((b, out_size), jnp.bfloat16),
            pltpu.VMEM((tile_m, out_size), jnp.float32),
        ],
    )(offs, sorted_x, W, pos_f)
